# baseline (speedup 1.0000x reference)
import functools

import jax
import jax.numpy as jnp
from jax import lax
from jax.experimental import pallas as pl
from jax.experimental.pallas import tpu as pltpu
from jax.experimental.pallas import tpu_sc as plsc

_NBUF = 3
_CHUNK_ROWS = 32


@functools.lru_cache(maxsize=None)
def _make_copy(n_seq, d_emb):
    info = plsc.get_sparse_core_info()
    nc, ns = info.num_cores, info.num_subcores
    nw = nc * ns
    rows_per_w = n_seq // nw
    n_chunks = rows_per_w // _CHUNK_ROWS

    mesh = plsc.VectorSubcoreMesh(core_axis_name="c", subcore_axis_name="s")

    @functools.partial(
        pl.kernel,
        mesh=mesh,
        out_type=jax.ShapeDtypeStruct((n_seq, d_emb), jnp.float32),
        scratch_types=[
            pltpu.VMEM((_NBUF, _CHUNK_ROWS, d_emb), jnp.float32),
            pltpu.SemaphoreType.DMA((_NBUF,)),
            pltpu.SemaphoreType.DMA((_NBUF,)),
        ],
    )
    def copy_kernel(table_hbm, out_hbm, bufs, rsems, wsems):
        wid = lax.axis_index("s") * nc + lax.axis_index("c")
        base = wid * rows_per_w

        def start_read(i):
            return pltpu.async_copy(
                table_hbm.at[pl.ds(base + i * _CHUNK_ROWS, _CHUNK_ROWS)],
                bufs.at[i % _NBUF],
                rsems.at[i % _NBUF],
            )

        def start_write(i):
            return pltpu.async_copy(
                bufs.at[i % _NBUF],
                out_hbm.at[pl.ds(base + i * _CHUNK_ROWS, _CHUNK_ROWS)],
                wsems.at[i % _NBUF],
            )

        reads = [None] * n_chunks
        writes = [None] * n_chunks
        reads[0] = start_read(0)
        for i in range(n_chunks):
            nxt = i + 1
            if nxt < n_chunks:
                if nxt >= _NBUF:
                    writes[nxt - _NBUF].wait()
                reads[nxt] = start_read(nxt)
            reads[i].wait()
            writes[i] = start_write(i)
        for i in range(max(0, n_chunks - _NBUF), n_chunks):
            writes[i].wait()

    return copy_kernel


def kernel(x, table):
    n_seq = x.shape[-1]
    return _make_copy(n_seq, table.shape[1])(table)


# speedup vs baseline: 1.0268x; 1.0268x over previous
import functools

import jax
import jax.numpy as jnp
from jax import lax
from jax.experimental import pallas as pl
from jax.experimental.pallas import tpu as pltpu
from jax.experimental.pallas import tpu_sc as plsc

_NBUF = 6
_CHUNK_ROWS = 16
_READ_AHEAD = 3


@functools.lru_cache(maxsize=None)
def _make_copy(n_seq, d_emb):
    info = plsc.get_sparse_core_info()
    nc, ns = info.num_cores, info.num_subcores
    nw = nc * ns
    rows_per_w = n_seq // nw
    n_chunks = rows_per_w // _CHUNK_ROWS

    mesh = plsc.VectorSubcoreMesh(core_axis_name="c", subcore_axis_name="s")

    @functools.partial(
        pl.kernel,
        mesh=mesh,
        out_type=jax.ShapeDtypeStruct((n_seq, d_emb), jnp.float32),
        scratch_types=[
            pltpu.VMEM((_NBUF, _CHUNK_ROWS, d_emb), jnp.float32),
            pltpu.SemaphoreType.DMA((_NBUF,)),
            pltpu.SemaphoreType.DMA((_NBUF,)),
        ],
    )
    def copy_kernel(table_hbm, out_hbm, bufs, rsems, wsems):
        wid = lax.axis_index("s") * nc + lax.axis_index("c")
        base = wid * rows_per_w

        def start_read(i):
            return pltpu.async_copy(
                table_hbm.at[pl.ds(base + i * _CHUNK_ROWS, _CHUNK_ROWS)],
                bufs.at[i % _NBUF],
                rsems.at[i % _NBUF],
            )

        def start_write(i):
            return pltpu.async_copy(
                bufs.at[i % _NBUF],
                out_hbm.at[pl.ds(base + i * _CHUNK_ROWS, _CHUNK_ROWS)],
                wsems.at[i % _NBUF],
            )

        reads = [None] * n_chunks
        writes = [None] * n_chunks
        for i in range(min(_READ_AHEAD, n_chunks)):
            reads[i] = start_read(i)
        for i in range(n_chunks):
            nxt = i + _READ_AHEAD
            if nxt < n_chunks:
                if nxt >= _NBUF:
                    writes[nxt - _NBUF].wait()
                reads[nxt] = start_read(nxt)
            reads[i].wait()
            writes[i] = start_write(i)
        for i in range(max(0, n_chunks - _NBUF), n_chunks):
            writes[i].wait()

    return copy_kernel


def kernel(x, table):
    n_seq = x.shape[-1]
    return _make_copy(n_seq, table.shape[1])(table)
